# baseline (speedup 1.0000x reference)
import jax
import jax.numpy as jnp
from jax.experimental import pallas as pl
import jax.experimental.pallas.tpu as pltpu
from jax.experimental.pallas import tpu_sc as plsc

C_BLK = 8
N_CELLS = 256
F_SLOW = 80
N_GENES = 2000
N_EMB = 100
N_BIAS = 128
LANES = 16


def _bias_gather_sc(gix_hbm, bias_hbm, brow_hbm, gix_v, bias_v, brow_v):
    cid = jax.lax.axis_index("c")
    sid = jax.lax.axis_index("s")

    @pl.when(jnp.logical_and(cid == 0, sid == 0))
    def _():
        pltpu.sync_copy(gix_hbm, gix_v)
        pltpu.sync_copy(bias_hbm, bias_v)

        def body(i, carry):
            idx = gix_v[pl.ds(i * LANES, LANES)]
            brow_v[pl.ds(i * LANES, LANES)] = plsc.load_gather(bias_v, [idx])
            return carry

        jax.lax.fori_loop(0, N_GENES // LANES, body, 0)
        pltpu.sync_copy(brow_v, brow_hbm)


def _mean_slow_kernel(emb_ref, brow_ref, out_ref):
    x = emb_ref[...]
    s = jnp.sum(x, axis=-1) * (1.0 / N_EMB)
    out_ref[...] = s + brow_ref[...]


def _mean_fast_kernel(emb_ref, brow_ref, out_ref):
    x = emb_ref[...]
    s = jnp.sum(x, axis=1) * (1.0 / N_EMB)
    out_ref[...] = s + brow_ref[...]


@jax.jit
def kernel(cell_gene_embedding, gene_ix, bias1):
    sc_gather = pl.kernel(
        _bias_gather_sc,
        mesh=plsc.VectorSubcoreMesh(core_axis_name="c", subcore_axis_name="s"),
        out_type=jax.ShapeDtypeStruct((N_GENES,), jnp.float32),
        scratch_types=[
            pltpu.VMEM((N_GENES,), jnp.int32),
            pltpu.VMEM((N_BIAS,), jnp.float32),
            pltpu.VMEM((N_GENES,), jnp.float32),
        ],
        compiler_params=pltpu.CompilerParams(needs_layout_passes=False),
    )
    brow = sc_gather(gene_ix.astype(jnp.int32), bias1).reshape(1, N_GENES)

    x_t = jnp.swapaxes(cell_gene_embedding[F_SLOW:], 1, 2)

    out_slow = pl.pallas_call(
        _mean_slow_kernel,
        grid=(F_SLOW // C_BLK,),
        in_specs=[
            pl.BlockSpec((C_BLK, N_GENES, N_EMB), lambda i: (i, 0, 0)),
            pl.BlockSpec((1, N_GENES), lambda i: (0, 0)),
        ],
        out_specs=pl.BlockSpec((C_BLK, N_GENES), lambda i: (i, 0)),
        out_shape=jax.ShapeDtypeStruct((F_SLOW, N_GENES), jnp.float32),
    )(cell_gene_embedding, brow)

    out_fast = pl.pallas_call(
        _mean_fast_kernel,
        grid=((N_CELLS - F_SLOW) // C_BLK,),
        in_specs=[
            pl.BlockSpec((C_BLK, N_EMB, N_GENES), lambda i: (i, 0, 0)),
            pl.BlockSpec((1, N_GENES), lambda i: (0, 0)),
        ],
        out_specs=pl.BlockSpec((C_BLK, N_GENES), lambda i: (i, 0)),
        out_shape=jax.ShapeDtypeStruct((N_CELLS - F_SLOW, N_GENES), jnp.float32),
    )(x_t, brow)

    return jnp.concatenate([out_slow, out_fast], axis=0)


# speedup vs baseline: 2.4162x; 2.4162x over previous
import jax
import jax.numpy as jnp
from jax.experimental import pallas as pl
import jax.experimental.pallas.tpu as pltpu
from jax.experimental.pallas import tpu_sc as plsc

C_BLK = 8
N_CELLS = 256
F_SLOW = 120
N_GENES = 2000
N_EMB = 100
N_BIAS = 128
LANES = 16


def _bias_gather_sc(gix_hbm, bias_hbm, brow_hbm, gix_v, bias_v, brow_v):
    cid = jax.lax.axis_index("c")
    sid = jax.lax.axis_index("s")

    @pl.when(jnp.logical_and(cid == 0, sid == 0))
    def _():
        pltpu.sync_copy(gix_hbm, gix_v)
        pltpu.sync_copy(bias_hbm, bias_v)

        def body(i, carry):
            idx = gix_v[pl.ds(i * LANES, LANES)]
            brow_v[pl.ds(i * LANES, LANES)] = plsc.load_gather(bias_v, [idx])
            return carry

        jax.lax.fori_loop(0, N_GENES // LANES, body, 0)
        pltpu.sync_copy(brow_v, brow_hbm)


def _mean_slow_kernel(emb_ref, brow_ref, out_ref):
    x = emb_ref[...]
    s = jnp.sum(x, axis=-1) * (1.0 / N_EMB)
    out_ref[...] = s + brow_ref[...]


def _mean_fast_kernel(emb_ref, brow_ref, out_ref):
    x = emb_ref[...]
    s = jnp.sum(x, axis=1) * (1.0 / N_EMB)
    out_ref[...] = s + brow_ref[...]


@jax.jit
def kernel(cell_gene_embedding, gene_ix, bias1):
    sc_gather = pl.kernel(
        _bias_gather_sc,
        mesh=plsc.VectorSubcoreMesh(core_axis_name="c", subcore_axis_name="s"),
        out_type=jax.ShapeDtypeStruct((N_GENES,), jnp.float32),
        scratch_types=[
            pltpu.VMEM((N_GENES,), jnp.int32),
            pltpu.VMEM((N_BIAS,), jnp.float32),
            pltpu.VMEM((N_GENES,), jnp.float32),
        ],
        compiler_params=pltpu.CompilerParams(needs_layout_passes=False),
    )
    brow = sc_gather(gene_ix.astype(jnp.int32), bias1).reshape(1, N_GENES)

    x_t = jnp.swapaxes(cell_gene_embedding, 1, 2)

    out_slow = pl.pallas_call(
        _mean_slow_kernel,
        grid=(F_SLOW // C_BLK,),
        in_specs=[
            pl.BlockSpec((C_BLK, N_GENES, N_EMB), lambda i: (i, 0, 0)),
            pl.BlockSpec((1, N_GENES), lambda i: (0, 0)),
        ],
        out_specs=pl.BlockSpec((C_BLK, N_GENES), lambda i: (i, 0)),
        out_shape=jax.ShapeDtypeStruct((F_SLOW, N_GENES), jnp.float32),
    )(cell_gene_embedding, brow)

    out_fast = pl.pallas_call(
        _mean_fast_kernel,
        grid=((N_CELLS - F_SLOW) // C_BLK,),
        in_specs=[
            pl.BlockSpec((C_BLK, N_EMB, N_GENES), lambda i: (i + F_SLOW // C_BLK, 0, 0)),
            pl.BlockSpec((1, N_GENES), lambda i: (0, 0)),
        ],
        out_specs=pl.BlockSpec((C_BLK, N_GENES), lambda i: (i, 0)),
        out_shape=jax.ShapeDtypeStruct((N_CELLS - F_SLOW, N_GENES), jnp.float32),
    )(x_t, brow)

    return jnp.concatenate([out_slow, out_fast], axis=0)


# speedup vs baseline: 4.8046x; 1.9886x over previous
import jax
import jax.numpy as jnp
from jax.experimental import pallas as pl
import jax.experimental.pallas.tpu as pltpu
from jax.experimental.pallas import tpu_sc as plsc

C_BLK = 8
N_CELLS = 256
N_GENES = 2000
N_EMB = 100
N_BIAS = 128
LANES = 16


def _bias_gather_sc(gix_hbm, bias_hbm, brow_hbm, gix_v, bias_v, brow_v):
    cid = jax.lax.axis_index("c")
    sid = jax.lax.axis_index("s")

    @pl.when(jnp.logical_and(cid == 0, sid == 0))
    def _():
        pltpu.sync_copy(gix_hbm, gix_v)
        pltpu.sync_copy(bias_hbm, bias_v)

        def body(i, carry):
            idx = gix_v[pl.ds(i * LANES, LANES)]
            brow_v[pl.ds(i * LANES, LANES)] = plsc.load_gather(bias_v, [idx])
            return carry

        jax.lax.fori_loop(0, N_GENES // LANES, body, 0)
        pltpu.sync_copy(brow_v, brow_hbm)


def _mean_kernel(emb_ref, brow_ref, out_ref):
    x = emb_ref[...]
    s = jnp.sum(x, axis=1) * (1.0 / N_EMB)
    out_ref[...] = s + brow_ref[...]


@jax.jit
def kernel(cell_gene_embedding, gene_ix, bias1):
    sc_gather = pl.kernel(
        _bias_gather_sc,
        mesh=plsc.VectorSubcoreMesh(core_axis_name="c", subcore_axis_name="s"),
        out_type=jax.ShapeDtypeStruct((N_GENES,), jnp.float32),
        scratch_types=[
            pltpu.VMEM((N_GENES,), jnp.int32),
            pltpu.VMEM((N_BIAS,), jnp.float32),
            pltpu.VMEM((N_GENES,), jnp.float32),
        ],
        compiler_params=pltpu.CompilerParams(needs_layout_passes=False),
    )
    brow = sc_gather(gene_ix.astype(jnp.int32), bias1)
    x_t = jnp.swapaxes(cell_gene_embedding, 1, 2)
    return pl.pallas_call(
        _mean_kernel,
        grid=(N_CELLS // C_BLK,),
        in_specs=[
            pl.BlockSpec((C_BLK, N_EMB, N_GENES), lambda i: (i, 0, 0)),
            pl.BlockSpec((1, N_GENES), lambda i: (0, 0)),
        ],
        out_specs=pl.BlockSpec((C_BLK, N_GENES), lambda i: (i, 0)),
        out_shape=jax.ShapeDtypeStruct((N_CELLS, N_GENES), jnp.float32),
    )(x_t, brow.reshape(1, N_GENES))
